# baseline (speedup 1.0000x reference)
import functools

import jax
import jax.numpy as jnp
from jax import lax
from jax.experimental import pallas as pl
from jax.experimental.pallas import tpu as pltpu
from jax.experimental.pallas import tpu_sc as plsc

NC, NS = 2, 16
NW = NC * NS
CHUNK = 96
DEGW = 128
BM = 1000
NBUF = 3
SPLIT0 = 80


def _mesh():
  return plsc.VectorSubcoreMesh(
      core_axis_name="c", subcore_axis_name="s", num_cores=NC, num_subcores=NS)


def _make_deg_kernel(npad, cpw, rpt):
  ngrp = cpw // 2

  @functools.partial(
      pl.kernel,
      out_type=jax.ShapeDtypeStruct((NC, npad, DEGW), jnp.float32),
      mesh=_mesh(),
      scratch_types=[
          pltpu.VMEM((2, CHUNK), jnp.int32),
          pltpu.VMEM((CHUNK, DEGW), jnp.float32),
          pltpu.VMEM_SHARED((npad, DEGW), jnp.float32),
          pltpu.SemaphoreType.DMA,
          pltpu.SemaphoreType.DMA,
      ],
  )
  def deg_kernel(dst_hbm, ones_hbm, zeros_hbm, out_hbm,
                 didx, ones_v, acc_sh, si0, si1):
    c = lax.axis_index("c")
    s = lax.axis_index("s")
    base = (s * NC + c) * cpw
    si = (si0, si1)

    pltpu.sync_copy(zeros_hbm, acc_sh.at[pl.ds(s * rpt, rpt)])
    pltpu.sync_copy(ones_hbm, ones_v)
    plsc.subcore_barrier()

    def load_idx(row, slot):
      pltpu.async_copy(dst_hbm.at[row], didx.at[slot], si[slot])

    def wait_idx(slot):
      pltpu.make_async_copy(dst_hbm.at[0], didx.at[slot], si[slot]).wait()

    def scatter(slot):
      pltpu.sync_copy(ones_v, acc_sh.at[didx.at[slot]], add=True)

    load_idx(base, 0)

    def body(gi, carry):
      j0 = base + 2 * gi
      wait_idx(0)
      load_idx(j0 + 1, 1)
      scatter(0)
      wait_idx(1)

      @pl.when(gi < ngrp - 1)
      def _():
        load_idx(j0 + 2, 0)

      scatter(1)
      return carry

    lax.fori_loop(0, ngrp, body, 0)
    plsc.subcore_barrier()
    pltpu.sync_copy(acc_sh.at[pl.ds(s * rpt, rpt)],
                    out_hbm.at[c, pl.ds(s * rpt, rpt)])

  return deg_kernel


def _make_edge_kernel(n, d, npad, k0, k1, rpt):
  assert k0 % NBUF == 0 and k1 % NBUF == 0
  lag = NBUF - 1

  @functools.partial(
      pl.kernel,
      out_type=jax.ShapeDtypeStruct((NC, npad, d), jnp.float32),
      mesh=_mesh(),
      scratch_types=[
          pltpu.VMEM((NBUF, CHUNK), jnp.int32),
          pltpu.VMEM((NBUF, CHUNK), jnp.int32),
          pltpu.VMEM((NBUF, CHUNK, d), jnp.float32),
          pltpu.VMEM_SHARED((npad, d), jnp.float32),
          [pltpu.SemaphoreType.DMA] * NBUF,
          [pltpu.SemaphoreType.DMA] * NBUF,
      ],
  )
  def edge_kernel(y_hbm, src_hbm, dst_hbm, zeros_hbm, out_hbm,
                  sidx, didx, rows, acc_sh, si, sg):
    c = lax.axis_index("c")
    s = lax.axis_index("s")
    base = jnp.where(c == 0, s * k0, NS * k0 + s * k1)
    ngrp = jnp.where(c == 0, k0 // NBUF, k1 // NBUF)

    pltpu.sync_copy(zeros_hbm, acc_sh.at[pl.ds(s * rpt, rpt)])
    plsc.subcore_barrier()

    def load_idx(row, slot):
      pltpu.async_copy(src_hbm.at[row], sidx.at[slot], si[slot])
      pltpu.async_copy(dst_hbm.at[row], didx.at[slot], si[slot])

    def wait_idx(slot):
      pltpu.make_async_copy(src_hbm.at[0], sidx.at[slot], si[slot]).wait()
      pltpu.make_async_copy(dst_hbm.at[0], didx.at[slot], si[slot]).wait()

    def start_gather(slot):
      pltpu.async_copy(y_hbm.at[sidx.at[slot]], rows.at[slot], sg[slot])

    def wait_gather(slot):
      pltpu.make_async_copy(y_hbm.at[sidx.at[slot]], rows.at[slot],
                            sg[slot]).wait()

    def scatter(slot):
      pltpu.sync_copy(rows.at[slot], acc_sh.at[didx.at[slot]], add=True)

    load_idx(base, 0)
    for b in range(lag):
      wait_idx(b)
      start_gather(b)
      load_idx(base + b + 1, b + 1)

    def body(gi, carry):
      j0 = base + NBUF * gi
      for b in range(NBUF):
        k = (b - lag) % NBUF

        def step(prefetch_next):
          wait_idx(b)
          start_gather(b)
          wait_gather(k)
          scatter(k)
          if prefetch_next:
            load_idx(j0 + b + 1, (b + 1) % NBUF)

        if b >= lag:
          step(prefetch_next=False)

          @pl.when(gi < ngrp - 1)
          def _():
            load_idx(j0 + b + 1, (b + 1) % NBUF)
        else:
          @pl.when(gi > 0)
          def _():
            step(prefetch_next=True)
      return carry

    lax.fori_loop(0, ngrp, body, 0)
    for b in range(lag):
      slot = (b - lag) % NBUF
      wait_gather(slot)
      scatter(slot)
    plsc.subcore_barrier()
    pltpu.sync_copy(acc_sh.at[pl.ds(s * rpt, rpt)],
                    out_hbm.at[c, pl.ds(s * rpt, rpt)])

  return edge_kernel


def _dis_col(degp_ref):
  deg = degp_ref[0] + degp_ref[1] + 1.0
  return lax.rsqrt(deg[:, 0:1])


def _mm1_body(x_ref, w_ref, degp_ref, y_ref):
  xw = jnp.dot(x_ref[...], w_ref[...], preferred_element_type=jnp.float32)
  y_ref[...] = xw * _dis_col(degp_ref)


def _mm2_body(accp_ref, y_ref, degp_ref, b_ref, w_ref, y2_ref):
  dis = _dis_col(degp_ref)
  h = (accp_ref[0] + accp_ref[1] + y_ref[...]) * dis + b_ref[...]
  h = jnp.maximum(h, 0.0)
  y2_ref[...] = jnp.dot(h, w_ref[...], preferred_element_type=jnp.float32) * dis


def _mm3_body(accp_ref, y_ref, degp_ref, b_ref, out_ref):
  dis = _dis_col(degp_ref)
  out_ref[...] = (accp_ref[0] + accp_ref[1] + y_ref[...]) * dis + b_ref[...]


def kernel(x, edge_index, W1, b1, W2, b2):
  n, d = x.shape
  e = edge_index.shape[1]
  npad = -(-(n + 1) // (NS * 8)) * (NS * 8)
  rpt = npad // NS
  cpw = -(-e // (NW * CHUNK))
  rnd = NBUF if NBUF % 2 == 0 else 2 * NBUF
  cpw = -(-cpw // rnd) * rnd
  epad = NW * cpw * CHUNK
  nchunks = NW * cpw
  k0 = (2 * cpw * SPLIT0 // 160) // NBUF * NBUF
  k1 = 2 * cpw - k0

  src_p = jnp.concatenate(
      [edge_index[0], jnp.zeros((epad - e,), jnp.int32)]).reshape(nchunks, CHUNK)
  dst_p = jnp.concatenate(
      [edge_index[1], jnp.full((epad - e,), n, jnp.int32)]).reshape(nchunks, CHUNK)
  zeros_deg = jnp.zeros((rpt, DEGW), jnp.float32)
  ones_deg = jnp.ones((CHUNK, DEGW), jnp.float32)
  zeros_acc = jnp.zeros((rpt, d), jnp.float32)

  deg_kernel = _make_deg_kernel(npad, cpw, rpt)
  edge_kernel = _make_edge_kernel(n, d, npad, k0, k1, rpt)

  grid = n // BM
  w_spec = pl.BlockSpec((d, d), lambda i: (0, 0))
  row_spec = pl.BlockSpec((BM, d), lambda i: (i, 0))
  degp_spec = pl.BlockSpec((NC, BM, DEGW), lambda i: (0, i, 0))
  accp_spec = pl.BlockSpec((NC, BM, d), lambda i: (0, i, 0))
  b_spec = pl.BlockSpec((1, d), lambda i: (0, 0))

  mm1 = pl.pallas_call(
      _mm1_body,
      grid=(grid,),
      in_specs=[row_spec, w_spec, degp_spec],
      out_specs=row_spec,
      out_shape=jax.ShapeDtypeStruct((n, d), jnp.float32),
  )
  mm2 = pl.pallas_call(
      _mm2_body,
      grid=(grid,),
      in_specs=[accp_spec, row_spec, degp_spec, b_spec, w_spec],
      out_specs=row_spec,
      out_shape=jax.ShapeDtypeStruct((n, d), jnp.float32),
  )
  mm3 = pl.pallas_call(
      _mm3_body,
      grid=(grid,),
      in_specs=[accp_spec, row_spec, degp_spec, b_spec],
      out_specs=row_spec,
      out_shape=jax.ShapeDtypeStruct((n, d), jnp.float32),
  )

  degp = deg_kernel(dst_p, ones_deg, zeros_deg)
  y1 = mm1(x, W1, degp)
  accp1 = edge_kernel(y1, src_p, dst_p, zeros_acc)
  y2 = mm2(accp1, y1, degp, b1.reshape(1, d), W2)
  accp2 = edge_kernel(y2, src_p, dst_p, zeros_acc)
  return mm3(accp2, y2, degp, b2.reshape(1, d))


# speedup vs baseline: 3.7948x; 3.7948x over previous
import functools

import jax
import jax.numpy as jnp
from jax import lax
from jax.experimental import pallas as pl
from jax.experimental.pallas import tpu as pltpu
from jax.experimental.pallas import tpu_sc as plsc

NC, NS = 2, 16
NW = NC * NS
CHUNK = 96
DEGW = 128
BM = 1000
NBUF = 3
SPLIT0 = 80


def _mesh():
  return plsc.VectorSubcoreMesh(
      core_axis_name="c", subcore_axis_name="s", num_cores=NC, num_subcores=NS)


def _make_deg_kernel(npad, cpw, rpt):
  ngrp = cpw // 2

  @functools.partial(
      pl.kernel,
      out_type=jax.ShapeDtypeStruct((NC, npad, DEGW), jnp.float32),
      mesh=_mesh(),
      scratch_types=[
          pltpu.VMEM((2, CHUNK), jnp.int32),
          pltpu.VMEM((CHUNK, DEGW), jnp.float32),
          pltpu.VMEM_SHARED((npad, DEGW), jnp.float32),
          pltpu.SemaphoreType.DMA,
          pltpu.SemaphoreType.DMA,
      ],
  )
  def deg_kernel(dst_hbm, ones_hbm, zeros_hbm, out_hbm,
                 didx, ones_v, acc_sh, si0, si1):
    c = lax.axis_index("c")
    s = lax.axis_index("s")
    base = (s * NC + c) * cpw
    si = (si0, si1)

    pltpu.sync_copy(zeros_hbm, acc_sh.at[pl.ds(s * rpt, rpt)])
    pltpu.sync_copy(ones_hbm, ones_v)
    plsc.subcore_barrier()

    def load_idx(row, slot):
      pltpu.async_copy(dst_hbm.at[row], didx.at[slot], si[slot])

    def wait_idx(slot):
      pltpu.make_async_copy(dst_hbm.at[0], didx.at[slot], si[slot]).wait()

    def scatter(slot):
      pltpu.sync_copy(ones_v, acc_sh.at[didx.at[slot]], add=True)

    load_idx(base, 0)

    def body(gi, carry):
      j0 = base + 2 * gi
      wait_idx(0)
      load_idx(j0 + 1, 1)
      scatter(0)
      wait_idx(1)

      @pl.when(gi < ngrp - 1)
      def _():
        load_idx(j0 + 2, 0)

      scatter(1)
      return carry

    lax.fori_loop(0, ngrp, body, 0)
    plsc.subcore_barrier()
    pltpu.sync_copy(acc_sh.at[pl.ds(s * rpt, rpt)],
                    out_hbm.at[c, pl.ds(s * rpt, rpt)])

  return deg_kernel


def _make_edge_kernel(n, d, npad, k0, k1, rpt):
  assert k0 % NBUF == 0 and k1 % NBUF == 0
  lag = NBUF - 1

  @functools.partial(
      pl.kernel,
      out_type=jax.ShapeDtypeStruct((NC, npad, d), jnp.float32),
      mesh=_mesh(),
      scratch_types=[
          pltpu.VMEM((NBUF, CHUNK), jnp.int32),
          pltpu.VMEM((NBUF, CHUNK), jnp.int32),
          pltpu.VMEM((NBUF, CHUNK, d), jnp.float32),
          pltpu.VMEM_SHARED((npad, d), jnp.float32),
          [pltpu.SemaphoreType.DMA] * NBUF,
          [pltpu.SemaphoreType.DMA] * NBUF,
      ],
  )
  def edge_kernel(y_hbm, src_hbm, dst_hbm, zeros_hbm, out_hbm,
                  sidx, didx, rows, acc_sh, si, sg):
    c = lax.axis_index("c")
    s = lax.axis_index("s")
    base = jnp.where(c == 0, s * k0, NS * k0 + s * k1)
    ngrp = jnp.where(c == 0, k0 // NBUF, k1 // NBUF)

    pltpu.sync_copy(zeros_hbm, acc_sh.at[pl.ds(s * rpt, rpt)])
    plsc.subcore_barrier()

    def load_idx(row, slot):
      pltpu.async_copy(src_hbm.at[row], sidx.at[slot], si[slot])
      pltpu.async_copy(dst_hbm.at[row], didx.at[slot], si[slot])

    def wait_idx(slot):
      pltpu.make_async_copy(src_hbm.at[0], sidx.at[slot], si[slot]).wait()
      pltpu.make_async_copy(dst_hbm.at[0], didx.at[slot], si[slot]).wait()

    def start_gather(slot):
      pltpu.async_copy(y_hbm.at[sidx.at[slot]], rows.at[slot], sg[slot])

    def wait_gather(slot):
      pltpu.make_async_copy(y_hbm.at[sidx.at[slot]], rows.at[slot],
                            sg[slot]).wait()

    def scatter(slot):
      pltpu.sync_copy(rows.at[slot], acc_sh.at[didx.at[slot]], add=True)

    load_idx(base, 0)
    for b in range(lag):
      wait_idx(b)
      start_gather(b)
      load_idx(base + b + 1, b + 1)

    def body(gi, carry):
      j0 = base + NBUF * gi
      for b in range(NBUF):
        k = (b - lag) % NBUF

        def step(prefetch_next):
          wait_idx(b)
          start_gather(b)
          wait_gather(k)
          scatter(k)
          if prefetch_next:
            load_idx(j0 + b + 1, (b + 1) % NBUF)

        if b >= lag:
          step(prefetch_next=False)

          @pl.when(gi < ngrp - 1)
          def _():
            load_idx(j0 + b + 1, (b + 1) % NBUF)
        else:
          @pl.when(gi > 0)
          def _():
            step(prefetch_next=True)
      return carry

    lax.fori_loop(0, ngrp, body, 0)
    for b in range(lag):
      slot = (b - lag) % NBUF
      wait_gather(slot)
      scatter(slot)
    plsc.subcore_barrier()
    pltpu.sync_copy(acc_sh.at[pl.ds(s * rpt, rpt)],
                    out_hbm.at[c, pl.ds(s * rpt, rpt)])

  return edge_kernel


def _dis_col(degp_ref):
  deg = degp_ref[0] + degp_ref[1] + 1.0
  return lax.rsqrt(deg[:, 0:1])


def _mm1_body(x_ref, w_ref, degp_ref, y_ref):
  xw = jnp.dot(x_ref[...], w_ref[...], preferred_element_type=jnp.float32)
  y_ref[...] = xw * _dis_col(degp_ref)


def _mm2_body(accp_ref, y_ref, degp_ref, b_ref, w_ref, y2_ref):
  dis = _dis_col(degp_ref)
  h = (accp_ref[0] + accp_ref[1] + y_ref[...]) * dis + b_ref[...]
  h = jnp.maximum(h, 0.0)
  y2_ref[...] = jnp.dot(h, w_ref[...], preferred_element_type=jnp.float32) * dis


def _mm3_body(accp_ref, y_ref, degp_ref, b_ref, out_ref):
  dis = _dis_col(degp_ref)
  out_ref[...] = (accp_ref[0] + accp_ref[1] + y_ref[...]) * dis + b_ref[...]


def kernel(x, edge_index, W1, b1, W2, b2):
  n, d = x.shape
  e = edge_index.shape[1]
  npad = -(-(n + 1) // (NS * 8)) * (NS * 8)
  rpt = npad // NS
  cpw = -(-e // (NW * CHUNK))
  rnd = NBUF if NBUF % 2 == 0 else 2 * NBUF
  cpw = -(-cpw // rnd) * rnd
  epad = NW * cpw * CHUNK
  nchunks = NW * cpw
  k0 = (2 * cpw * SPLIT0 // 160) // NBUF * NBUF
  k1 = 2 * cpw - k0

  pad_ids = jnp.arange(epad - e, dtype=jnp.int32)
  src_p = jnp.concatenate(
      [edge_index[0], pad_ids % jnp.int32(n)]).reshape(nchunks, CHUNK)
  dst_p = jnp.concatenate(
      [edge_index[1], n + pad_ids % jnp.int32(npad - n)]).reshape(nchunks, CHUNK)
  zeros_deg = jnp.zeros((rpt, DEGW), jnp.float32)
  ones_deg = jnp.ones((CHUNK, DEGW), jnp.float32)
  zeros_acc = jnp.zeros((rpt, d), jnp.float32)

  deg_kernel = _make_deg_kernel(npad, cpw, rpt)
  edge_kernel = _make_edge_kernel(n, d, npad, k0, k1, rpt)

  grid = n // BM
  w_spec = pl.BlockSpec((d, d), lambda i: (0, 0))
  row_spec = pl.BlockSpec((BM, d), lambda i: (i, 0))
  degp_spec = pl.BlockSpec((NC, BM, DEGW), lambda i: (0, i, 0))
  accp_spec = pl.BlockSpec((NC, BM, d), lambda i: (0, i, 0))
  b_spec = pl.BlockSpec((1, d), lambda i: (0, 0))

  mm1 = pl.pallas_call(
      _mm1_body,
      grid=(grid,),
      in_specs=[row_spec, w_spec, degp_spec],
      out_specs=row_spec,
      out_shape=jax.ShapeDtypeStruct((n, d), jnp.float32),
  )
  mm2 = pl.pallas_call(
      _mm2_body,
      grid=(grid,),
      in_specs=[accp_spec, row_spec, degp_spec, b_spec, w_spec],
      out_specs=row_spec,
      out_shape=jax.ShapeDtypeStruct((n, d), jnp.float32),
  )
  mm3 = pl.pallas_call(
      _mm3_body,
      grid=(grid,),
      in_specs=[accp_spec, row_spec, degp_spec, b_spec],
      out_specs=row_spec,
      out_shape=jax.ShapeDtypeStruct((n, d), jnp.float32),
  )

  degp = deg_kernel(dst_p, ones_deg, zeros_deg)
  y1 = mm1(x, W1, degp)
  accp1 = edge_kernel(y1, src_p, dst_p, zeros_acc)
  y2 = mm2(accp1, y1, degp, b1.reshape(1, d), W2)
  accp2 = edge_kernel(y2, src_p, dst_p, zeros_acc)
  return mm3(accp2, y2, degp, b2.reshape(1, d))
